# baseline (speedup 1.0000x reference)
import jax
import jax.numpy as jnp
from jax.experimental import pallas as pl
from jax.experimental.pallas import tpu as pltpu

K = 100
C = 1024
HW = 1024
KP = 128
KH = 50
KHP = 64
CCH = 128
NCH = C // CCH


def _ranks_to_onehot(vcol, vrow):
    ii = jax.lax.broadcasted_iota(jnp.int32, (HW, HW), 1)
    jj = jax.lax.broadcasted_iota(jnp.int32, (HW, HW), 0)
    gt = vrow > vcol
    tie = (vrow == vcol) & (ii < jj)
    rank_col = jnp.sum(jnp.where(gt | tie, 1, 0), axis=1, keepdims=True)
    kio = jax.lax.broadcasted_iota(jnp.int32, (HW, KP), 1)
    return jnp.where((rank_col == kio) & (kio < K), 1.0, 0.0)


def _branch_finalize(raw, pt, relu_cwf_row, colnorm2_row, m_row,
                     fc1w, fc1b, fc2w, fc2b):
    x_col = jax.lax.dot_general(pt, relu_cwf_row, (((0,), (1,)), ((), ())),
                                preferred_element_type=jnp.float32)
    h = jax.lax.dot_general(fc1w, x_col, (((1,), (0,)), ((), ())),
                            preferred_element_type=jnp.float32) + fc1b
    y = jax.lax.dot_general(fc2w, h, (((1,), (0,)), ((), ())),
                            preferred_element_type=jnp.float32) + fc2b
    gate_col = jax.nn.sigmoid(y)

    pn2_col = jax.lax.dot_general(pt, colnorm2_row, (((0,), (1,)), ((), ())),
                                  preferred_element_type=jnp.float32)
    pn_col = jnp.sqrt(gate_col * gate_col * pn2_col + 1e-12)
    on_row = jnp.sqrt(colnorm2_row * m_row + 1e-12)

    att = (raw * m_row) * gate_col / (pn_col * on_row + 1e-8)
    att = jnp.maximum(att, 0.0)
    return jnp.max(att, axis=0, keepdims=True)


def _body(feats_ref, soft_ref, soft_t_ref,
          cw_f_ref, fc1w_f_ref, fc1b_f_ref, fc2w_f_ref, fc2b_f_ref,
          cw_b_ref, fc1w_b_ref, fc1b_b_ref, fc2w_b_ref, fc2b_b_ref,
          out_ref,
          feats2_ref, ptf_ref, ptb_ref, rawf_ref, rawb_ref,
          cwf_ref, cwb_ref, cn2_ref, factor_ref):
    j = pl.program_id(1)

    @pl.when(j == 0)
    def _init():
        soft_t = soft_t_ref[0]
        soft = soft_ref[0]
        ptf_ref[...] = _ranks_to_onehot(soft_t[:, 1:2], soft[1:2, :])
        ptb_ref[...] = _ranks_to_onehot(soft_t[:, 0:1], soft[0:1, :])
        rawf_ref[...] = jnp.zeros((KP, HW), jnp.float32)
        rawb_ref[...] = jnp.zeros((KP, HW), jnp.float32)
        cwf_ref[...] = jnp.zeros((1, HW), jnp.float32)
        cwb_ref[...] = jnp.zeros((1, HW), jnp.float32)
        cn2_ref[...] = jnp.zeros((1, HW), jnp.float32)

    @pl.when(j < NCH)
    def _accum():
        c0 = j * CCH
        chunk = jnp.reshape(feats_ref[0], (CCH, HW))
        feats2_ref[pl.ds(c0, CCH), :] = chunk
        cn2_ref[...] += jnp.sum(chunk * chunk, axis=0, keepdims=True)
        cwf_ref[...] += jax.lax.dot_general(
            cw_f_ref[:, pl.ds(c0, CCH)], chunk, (((1,), (0,)), ((), ())),
            preferred_element_type=jnp.float32)
        cwb_ref[...] += jax.lax.dot_general(
            cw_b_ref[:, pl.ds(c0, CCH)], chunk, (((1,), (0,)), ((), ())),
            preferred_element_type=jnp.float32)
        pch_f = jnp.dot(chunk, ptf_ref[...], preferred_element_type=jnp.float32)
        pch_b = jnp.dot(chunk, ptb_ref[...], preferred_element_type=jnp.float32)
        rawf_ref[...] += jax.lax.dot_general(
            pch_f, chunk, (((0,), (0,)), ((), ())),
            preferred_element_type=jnp.float32)
        rawb_ref[...] += jax.lax.dot_general(
            pch_b, chunk, (((0,), (0,)), ((), ())),
            preferred_element_type=jnp.float32)

    @pl.when(j == NCH)
    def _finalize():
        soft = soft_ref[0]
        s0r = soft[0:1, :]
        s1r = soft[1:2, :]
        fg_row = jnp.where(s1r > s0r, 1.0, 0.0)
        bg_row = 1.0 - fg_row
        cn2 = cn2_ref[...]
        fore = _branch_finalize(rawf_ref[...], ptf_ref[...],
                                jnp.maximum(cwf_ref[...], 0.0), cn2, bg_row,
                                fc1w_f_ref[...], fc1b_f_ref[...],
                                fc2w_f_ref[...], fc2b_f_ref[...])
        back = _branch_finalize(rawb_ref[...], ptb_ref[...],
                                jnp.maximum(cwb_ref[...], 0.0), cn2, fg_row,
                                fc1w_b_ref[...], fc1b_b_ref[...],
                                fc2w_b_ref[...], fc2b_b_ref[...])
        factor_ref[...] = 1.0 + s1r - back + fore

    @pl.when(j >= NCH)
    def _emit():
        c0 = (j - NCH) * CCH
        out2 = feats2_ref[pl.ds(c0, CCH), :] * factor_ref[...]
        out_ref[0] = jnp.reshape(out2, (CCH, 32, 32))


def _pad2(a, r, c):
    out = jnp.zeros((r, c), a.dtype)
    return out.at[:a.shape[0], :a.shape[1]].set(a)


def kernel(feats, soft_mask, conv_w_f, fc1_w_f, fc1_b_f, fc2_w_f, fc2_b_f,
           conv_w_b, fc1_w_b, fc1_b_b, fc2_w_b, fc2_b_b):
    b, c, h, w = feats.shape
    hw = h * w
    soft3 = soft_mask.reshape(b, 2, hw)
    soft3_t = jnp.transpose(soft3, (0, 2, 1))

    args = (
        feats, soft3, soft3_t,
        conv_w_f.reshape(1, c),
        _pad2(fc1_w_f, KHP, KP), _pad2(fc1_b_f.reshape(KH, 1), KHP, 1),
        _pad2(fc2_w_f, KP, KHP), _pad2(fc2_b_f.reshape(K, 1), KP, 1),
        conv_w_b.reshape(1, c),
        _pad2(fc1_w_b, KHP, KP), _pad2(fc1_b_b.reshape(KH, 1), KHP, 1),
        _pad2(fc2_w_b, KP, KHP), _pad2(fc2_b_b.reshape(K, 1), KP, 1),
    )

    def fixed(shape):
        return pl.BlockSpec(shape, lambda i, j: (0,) * len(shape))

    out = pl.pallas_call(
        _body,
        grid=(b, 2 * NCH),
        in_specs=[
            pl.BlockSpec((1, CCH, h, w),
                         lambda i, j: (i, jnp.minimum(j, NCH - 1), 0, 0)),
            pl.BlockSpec((1, 2, hw), lambda i, j: (i, 0, 0)),
            pl.BlockSpec((1, hw, 2), lambda i, j: (i, 0, 0)),
            fixed((1, c)),
            fixed((KHP, KP)), fixed((KHP, 1)),
            fixed((KP, KHP)), fixed((KP, 1)),
            fixed((1, c)),
            fixed((KHP, KP)), fixed((KHP, 1)),
            fixed((KP, KHP)), fixed((KP, 1)),
        ],
        out_specs=pl.BlockSpec((1, CCH, h, w),
                               lambda i, j: (i, jnp.maximum(j - NCH, 0), 0, 0)),
        out_shape=jax.ShapeDtypeStruct((b, c, h, w), jnp.float32),
        scratch_shapes=[
            pltpu.VMEM((C, HW), jnp.float32),
            pltpu.VMEM((HW, KP), jnp.float32),
            pltpu.VMEM((HW, KP), jnp.float32),
            pltpu.VMEM((KP, HW), jnp.float32),
            pltpu.VMEM((KP, HW), jnp.float32),
            pltpu.VMEM((1, HW), jnp.float32),
            pltpu.VMEM((1, HW), jnp.float32),
            pltpu.VMEM((1, HW), jnp.float32),
            pltpu.VMEM((1, HW), jnp.float32),
        ],
    )(*args)
    return out


# speedup vs baseline: 4.0926x; 4.0926x over previous
import jax
import jax.numpy as jnp
from jax.experimental import pallas as pl

K = 100
C = 1024
HW = 1024
KP = 128
KH = 50
KHP = 64


def _onehot_topk(vcol, vrow):
    ii = jax.lax.broadcasted_iota(jnp.int32, (HW, HW), 1)
    jj = jax.lax.broadcasted_iota(jnp.int32, (HW, HW), 0)
    gt = vrow > vcol
    tie = (vrow == vcol) & (ii < jj)
    rank_col = jnp.sum(jnp.where(gt | tie, 1, 0), axis=1, keepdims=True)
    kio = jax.lax.broadcasted_iota(jnp.int32, (HW, KP), 1)
    return jnp.where((rank_col == kio) & (kio < K), 1.0, 0.0)


def _branch(ft, rn2_col, vcol, vrow, m_col, cw_col,
            fc1wt, fc1b, fc2wt, fc2b):
    pt = _onehot_topk(vcol, vrow)

    cwf_col = jax.lax.dot_general(ft, cw_col, (((1,), (0,)), ((), ())),
                                  preferred_element_type=jnp.float32)
    relu_cwf = jnp.maximum(cwf_col, 0.0)
    x_row = jax.lax.dot_general(relu_cwf, pt, (((0,), (0,)), ((), ())),
                                preferred_element_type=jnp.float32)
    h_row = jax.lax.dot_general(x_row, fc1wt, (((1,), (0,)), ((), ())),
                                preferred_element_type=jnp.float32) + fc1b
    y_row = jax.lax.dot_general(h_row, fc2wt, (((1,), (0,)), ((), ())),
                                preferred_element_type=jnp.float32) + fc2b
    gate_row = jax.nn.sigmoid(y_row)

    pn2_row = jax.lax.dot_general(rn2_col, pt, (((0,), (0,)), ((), ())),
                                  preferred_element_type=jnp.float32)
    pn_row = jnp.sqrt(gate_row * gate_row * pn2_row + 1e-12)
    on_col = jnp.sqrt(rn2_col * m_col + 1e-12)

    protos = jax.lax.dot_general(pt, ft, (((0,), (0,)), ((), ())),
                                 preferred_element_type=jnp.float32)
    rawt = jax.lax.dot_general(ft, protos, (((1,), (1,)), ((), ())),
                               preferred_element_type=jnp.float32)
    att = (rawt * m_col) * gate_row / (on_col * pn_row + 1e-8)
    att = jnp.maximum(att, 0.0)
    return jnp.max(att, axis=1, keepdims=True)


def _body(ft_ref, soft_ref, soft_t_ref,
          cw_f_ref, fc1wt_f_ref, fc1b_f_ref, fc2wt_f_ref, fc2b_f_ref,
          cw_b_ref, fc1wt_b_ref, fc1b_b_ref, fc2wt_b_ref, fc2b_b_ref,
          out_ref):
    ft = ft_ref[0]
    soft = soft_ref[0]
    soft_t = soft_t_ref[0]
    s0c = soft_t[:, 0:1]
    s1c = soft_t[:, 1:2]
    fg_col = jnp.where(s1c > s0c, 1.0, 0.0)
    bg_col = 1.0 - fg_col

    rn2_col = jnp.sum(ft * ft, axis=1, keepdims=True)

    fore = _branch(ft, rn2_col, s1c, soft[1:2, :], bg_col, cw_f_ref[...],
                   fc1wt_f_ref[...], fc1b_f_ref[...],
                   fc2wt_f_ref[...], fc2b_f_ref[...])
    back = _branch(ft, rn2_col, s0c, soft[0:1, :], fg_col, cw_b_ref[...],
                   fc1wt_b_ref[...], fc1b_b_ref[...],
                   fc2wt_b_ref[...], fc2b_b_ref[...])

    out_ref[0] = ft * (1.0 + s1c - back + fore)


def _pad2(a, r, c):
    out = jnp.zeros((r, c), a.dtype)
    return out.at[:a.shape[0], :a.shape[1]].set(a)


def kernel(feats, soft_mask, conv_w_f, fc1_w_f, fc1_b_f, fc2_w_f, fc2_b_f,
           conv_w_b, fc1_w_b, fc1_b_b, fc2_w_b, fc2_b_b):
    b, c, h, w = feats.shape
    hw = h * w
    ft3 = jnp.transpose(feats, (0, 2, 3, 1)).reshape(b, hw, c)
    soft3 = soft_mask.reshape(b, 2, hw)
    soft3_t = jnp.transpose(soft3, (0, 2, 1))

    args = (
        ft3, soft3, soft3_t,
        conv_w_f.reshape(c, 1),
        _pad2(fc1_w_f.T, KP, KHP), _pad2(fc1_b_f.reshape(1, KH), 1, KHP),
        _pad2(fc2_w_f.T, KHP, KP), _pad2(fc2_b_f.reshape(1, K), 1, KP),
        conv_w_b.reshape(c, 1),
        _pad2(fc1_w_b.T, KP, KHP), _pad2(fc1_b_b.reshape(1, KH), 1, KHP),
        _pad2(fc2_w_b.T, KHP, KP), _pad2(fc2_b_b.reshape(1, K), 1, KP),
    )

    def fixed(shape):
        return pl.BlockSpec(shape, lambda i: (0,) * len(shape))

    out_t = pl.pallas_call(
        _body,
        grid=(b,),
        in_specs=[
            pl.BlockSpec((1, hw, c), lambda i: (i, 0, 0)),
            pl.BlockSpec((1, 2, hw), lambda i: (i, 0, 0)),
            pl.BlockSpec((1, hw, 2), lambda i: (i, 0, 0)),
            fixed((c, 1)),
            fixed((KP, KHP)), fixed((1, KHP)),
            fixed((KHP, KP)), fixed((1, KP)),
            fixed((c, 1)),
            fixed((KP, KHP)), fixed((1, KHP)),
            fixed((KHP, KP)), fixed((1, KP)),
        ],
        out_specs=pl.BlockSpec((1, hw, c), lambda i: (i, 0, 0)),
        out_shape=jax.ShapeDtypeStruct((b, hw, c), jnp.float32),
    )(*args)
    return jnp.transpose(out_t.reshape(b, h, w, c), (0, 3, 1, 2))


# speedup vs baseline: 7.5897x; 1.8545x over previous
import jax
import jax.numpy as jnp
from jax.experimental import pallas as pl

K = 100
C = 1024
HW = 1024
KH = 50


def _branch(ft, rn2_col, vcol, vrow, m_col, cw_row, fc1w, fc1b, fc2w, fc2b,
            ii, jj):
    gt = vrow > vcol
    tie = (vrow == vcol) & (ii < jj)
    rank_col = jnp.sum(jnp.where(gt | tie, 1, 0), axis=1, keepdims=True)
    kio = jax.lax.broadcasted_iota(jnp.int32, (HW, K), 1)
    pt = jnp.where(rank_col == kio, 1.0, 0.0)

    cwf_col = jax.lax.dot_general(ft, cw_row, (((1,), (1,)), ((), ())),
                                  preferred_element_type=jnp.float32)
    relu_cwf = jnp.maximum(cwf_col, 0.0)
    x_row = jax.lax.dot_general(relu_cwf, pt, (((0,), (0,)), ((), ())),
                                preferred_element_type=jnp.float32)
    h_row = jax.lax.dot_general(x_row, fc1w, (((1,), (1,)), ((), ())),
                                preferred_element_type=jnp.float32) + fc1b
    y_row = jax.lax.dot_general(h_row, fc2w, (((1,), (1,)), ((), ())),
                                preferred_element_type=jnp.float32) + fc2b
    gate_row = jax.nn.sigmoid(y_row)

    pn2_row = jax.lax.dot_general(rn2_col, pt, (((0,), (0,)), ((), ())),
                                  preferred_element_type=jnp.float32)
    pn_row = jnp.sqrt(gate_row * gate_row * pn2_row + 1e-12)
    on_col = jnp.sqrt(rn2_col * m_col + 1e-12)

    protos = jax.lax.dot_general(pt, ft, (((0,), (0,)), ((), ())),
                                 preferred_element_type=jnp.float32)
    rawt = jax.lax.dot_general(ft, protos, (((1,), (1,)), ((), ())),
                               preferred_element_type=jnp.float32)
    att = (rawt * m_col) * gate_row / (on_col * pn_row + 1e-8)
    att = jnp.maximum(att, 0.0)
    return jnp.max(att, axis=1, keepdims=True)


def _body(ft_ref, soft_ref,
          cw_f_ref, fc1w_f_ref, fc1b_f_ref, fc2w_f_ref, fc2b_f_ref,
          cw_b_ref, fc1w_b_ref, fc1b_b_ref, fc2w_b_ref, fc2b_b_ref,
          out_ref):
    ft = ft_ref[0]
    soft = jnp.reshape(soft_ref[0], (2, HW))
    s0r = soft[0:1, :]
    s1r = soft[1:2, :]

    ii = jax.lax.broadcasted_iota(jnp.int32, (HW, HW), 1)
    jj = jax.lax.broadcasted_iota(jnp.int32, (HW, HW), 0)
    ident = jnp.where(ii == jj, 1.0, 0.0)
    s0c = jax.lax.dot_general(ident, s0r, (((1,), (1,)), ((), ())),
                              preferred_element_type=jnp.float32)
    s1c = jax.lax.dot_general(ident, s1r, (((1,), (1,)), ((), ())),
                              preferred_element_type=jnp.float32)

    fg_col = jnp.where(s1c > s0c, 1.0, 0.0)
    bg_col = 1.0 - fg_col

    rn2_col = jnp.sum(ft * ft, axis=1, keepdims=True)

    fore = _branch(ft, rn2_col, s1c, s1r, bg_col,
                   jnp.reshape(cw_f_ref[...], (1, C)),
                   fc1w_f_ref[...], jnp.reshape(fc1b_f_ref[...], (1, KH)),
                   fc2w_f_ref[...], jnp.reshape(fc2b_f_ref[...], (1, K)),
                   ii, jj)
    back = _branch(ft, rn2_col, s0c, s0r, fg_col,
                   jnp.reshape(cw_b_ref[...], (1, C)),
                   fc1w_b_ref[...], jnp.reshape(fc1b_b_ref[...], (1, KH)),
                   fc2w_b_ref[...], jnp.reshape(fc2b_b_ref[...], (1, K)),
                   ii, jj)

    out_ref[0] = ft * (1.0 + s1c - back + fore)


def kernel(feats, soft_mask, conv_w_f, fc1_w_f, fc1_b_f, fc2_w_f, fc2_b_f,
           conv_w_b, fc1_w_b, fc1_b_b, fc2_w_b, fc2_b_b):
    b, c, h, w = feats.shape
    hw = h * w
    ft3 = jnp.transpose(feats, (0, 2, 3, 1)).reshape(b, hw, c)

    args = (
        ft3, soft_mask,
        conv_w_f, fc1_w_f, fc1_b_f, fc2_w_f, fc2_b_f,
        conv_w_b, fc1_w_b, fc1_b_b, fc2_w_b, fc2_b_b,
    )

    def fixed(shape):
        return pl.BlockSpec(shape, lambda i: (0,) * len(shape))

    out_t = pl.pallas_call(
        _body,
        grid=(b,),
        in_specs=[
            pl.BlockSpec((1, hw, c), lambda i: (i, 0, 0)),
            pl.BlockSpec((1, 2, h, w), lambda i: (i, 0, 0, 0)),
            fixed((c,)),
            fixed((KH, K)), fixed((KH,)), fixed((K, KH)), fixed((K,)),
            fixed((c,)),
            fixed((KH, K)), fixed((KH,)), fixed((K, KH)), fixed((K,)),
        ],
        out_specs=pl.BlockSpec((1, hw, c), lambda i: (i, 0, 0)),
        out_shape=jax.ShapeDtypeStruct((b, hw, c), jnp.float32),
    )(*args)
    return jnp.transpose(out_t.reshape(b, h, w, c), (0, 3, 1, 2))
